# baseline (speedup 1.0000x reference)
import functools

import jax
import jax.numpy as jnp
from jax import lax
from jax.experimental import pallas as pl
from jax.experimental.pallas import tpu as pltpu
from jax.experimental.pallas import tpu_sc as plsc

B, E, C, O = 8, 1024, 32, 8192
N = B * E
TOK_BLK = 1024
CB_CHUNK = 2048
N_BLKS = N // TOK_BLK
N_CHUNKS = O // CB_CHUNK

_NW = 32
_GROUPS = (2 * N) // 128
_RPW = _GROUPS // _NW


def _argmin_body(x_ref, w_ref, b_ref, cbt_ref, noise_ref, std_ref,
                 lat_ref, idet_ref, inoisy_ref, lat_s, bv_s, bi_s):
    k = pl.program_id(1)

    @pl.when(k == 0)
    def _init():
        lat0 = (jnp.dot(x_ref[...], w_ref[...],
                        preferred_element_type=jnp.float32)
                + b_ref[...][None, :])
        lat_s[...] = lat0
        lat_ref[...] = lat0
        bv_s[...] = jnp.full((TOK_BLK,), jnp.inf, jnp.float32)
        bi_s[...] = jnp.zeros((TOK_BLK,), jnp.int32)

    lat = lat_s[...]
    cbt = cbt_ref[...]
    f_sq = jnp.sum(lat * lat, axis=1)
    cb_sq = jnp.sum(cbt * cbt, axis=0)
    prod = jnp.dot(lat.astype(jnp.bfloat16), cbt.astype(jnp.bfloat16),
                   preferred_element_type=jnp.float32)
    scores = (f_sq[:, None] + cb_sq[None, :]) - 2.0 * prod
    m = jnp.min(scores, axis=1)
    cols = (lax.broadcasted_iota(jnp.int32, scores.shape, 1)
            + k * CB_CHUNK)
    idx = jnp.min(jnp.where(scores == m[:, None], cols, O), axis=1)
    take = m < bv_s[...]
    bi_s[...] = jnp.where(take, idx, bi_s[...])
    bv_s[...] = (jnp.where(take, m, bv_s[...])
                 .astype(jnp.bfloat16).astype(jnp.float32))

    @pl.when(k == N_CHUNKS - 1)
    def _fin():
        best_i = bi_s[...]
        idet_ref[...] = best_i
        nz = jnp.round(noise_ref[...] * std_ref[0, 0]).astype(jnp.int32)
        inoisy_ref[...] = jnp.clip(best_i + nz, 0, O - 1)


def _argmin_call(x2, W_pre, b_pre, cbT, noise_f, std_arr):
    return pl.pallas_call(
        _argmin_body,
        grid=(N_BLKS, N_CHUNKS),
        in_specs=[
            pl.BlockSpec((TOK_BLK, C), lambda i, k: (i, 0)),
            pl.BlockSpec((C, C), lambda i, k: (0, 0)),
            pl.BlockSpec((C,), lambda i, k: (0,)),
            pl.BlockSpec((C, CB_CHUNK), lambda i, k: (0, k)),
            pl.BlockSpec((TOK_BLK,), lambda i, k: (i,)),
            pl.BlockSpec(memory_space=pltpu.SMEM),
        ],
        out_specs=[
            pl.BlockSpec((TOK_BLK, C), lambda i, k: (i, 0)),
            pl.BlockSpec((TOK_BLK,), lambda i, k: (i,)),
            pl.BlockSpec((TOK_BLK,), lambda i, k: (i,)),
        ],
        out_shape=[
            jax.ShapeDtypeStruct((N, C), jnp.float32),
            jax.ShapeDtypeStruct((N,), jnp.int32),
            jax.ShapeDtypeStruct((N,), jnp.int32),
        ],
        scratch_shapes=[
            pltpu.VMEM((TOK_BLK, C), jnp.float32),
            pltpu.VMEM((TOK_BLK,), jnp.float32),
            pltpu.VMEM((TOK_BLK,), jnp.int32),
        ],
    )(x2, W_pre, b_pre, cbT, noise_f, std_arr)


@functools.cache
def _sc_gather_fn():
    mesh = plsc.VectorSubcoreMesh(core_axis_name="c", subcore_axis_name="s",
                                  num_cores=2)

    @functools.partial(
        pl.kernel,
        mesh=mesh,
        out_type=jax.ShapeDtypeStruct((_GROUPS, 128, C), jnp.float32),
        scratch_types=[
            pltpu.VMEM((_RPW, 128), jnp.int32),
            pltpu.VMEM((_RPW, 128, C), jnp.float32),
            pltpu.SemaphoreType.DMA,
        ],
        compiler_params=pltpu.CompilerParams(use_tc_tiling_on_sc=False),
    )
    def _sc_gather(idx_hbm, table_hbm, out_hbm, idx_v, rows_v, sem):
        wid = lax.axis_index("s") * 2 + lax.axis_index("c")
        base = wid * _RPW
        pltpu.sync_copy(idx_hbm.at[pl.ds(base, _RPW)], idx_v)
        cps = [pltpu.async_copy(table_hbm.at[idx_v.at[j]], rows_v.at[j], sem)
               for j in range(_RPW)]
        for cp in cps:
            cp.wait()
        pltpu.sync_copy(rows_v, out_hbm.at[pl.ds(base, _RPW)])

    return _sc_gather


def _post_body(lat_ref, qd_ref, qn_ref, y_ref, w_ref, b_ref,
               out_ref, loss_ref):
    lat = lat_ref[...]
    qd = qd_ref[...]
    qn = qn_ref[...]
    out_in = lat + (qn - lat)
    out = (jnp.dot(out_in, w_ref[...], preferred_element_type=jnp.float32)
           + b_ref[...][None, :])
    out_ref[...] = out
    inv = jnp.float32(1.0 / (N * C))
    recon = jnp.sum((out - y_ref[...]) ** 2) * inv
    non_recon = (jnp.float32(0.25) * (jnp.sum((lat - qd) ** 2) * inv)
                 + jnp.sum((qn - lat) ** 2) * inv)
    loss_ref[0, 0] = recon + non_recon


def _post_call(lat, q_det, q_noisy, y2, W_post, b_post):
    return pl.pallas_call(
        _post_body,
        grid=(1,),
        in_specs=[
            pl.BlockSpec((N, C), lambda i: (0, 0)),
            pl.BlockSpec((N, C), lambda i: (0, 0)),
            pl.BlockSpec((N, C), lambda i: (0, 0)),
            pl.BlockSpec((N, C), lambda i: (0, 0)),
            pl.BlockSpec((C, C), lambda i: (0, 0)),
            pl.BlockSpec((C,), lambda i: (0,)),
        ],
        out_specs=[
            pl.BlockSpec((N, C), lambda i: (0, 0)),
            pl.BlockSpec(memory_space=pltpu.SMEM),
        ],
        out_shape=[
            jax.ShapeDtypeStruct((N, C), jnp.float32),
            jax.ShapeDtypeStruct((1, 1), jnp.float32),
        ],
    )(lat, q_det, q_noisy, y2, W_post, b_post)


def kernel(x, y, quantization_noise_std, W_pre, b_pre, W_post, b_post,
           codebook):
    x2 = x.reshape(N, C)
    y2 = y.reshape(N, C)
    std_arr = jnp.asarray(quantization_noise_std, jnp.float32).reshape(1, 1)
    noise_f = jax.random.normal(jax.random.key(42), (N,), dtype=jnp.float32)

    lat, idet, inoisy = _argmin_call(x2, W_pre, b_pre, codebook.T, noise_f,
                                     std_arr)

    idx_mat = jnp.concatenate([idet, inoisy]).reshape(_GROUPS, 128)
    rows = _sc_gather_fn()(idx_mat, codebook).reshape(2 * N, C)
    q_det, q_noisy = rows[:N], rows[N:]

    out2, loss = _post_call(lat, q_det, q_noisy, y2, W_post, b_post)
    return out2.reshape(B, E, C), loss[0, 0]
